# Initial kernel scaffold; baseline (speedup 1.0000x reference)
import jax
import jax.numpy as jnp
from jax.experimental import pallas as pl


def kernel(node_input, edge_input, edge_index, batch, W1, b1, W2, b2):
    raise NotImplementedError("write your pallas kernel here")


# speedup vs baseline: 52.0748x; 52.0748x over previous
import dataclasses
import functools

import jax
import jax.numpy as jnp
from jax import lax
from jax.experimental import pallas as pl
from jax.experimental.pallas import tpu as pltpu
from jax.experimental.pallas import tpu_sc as plsc

_G = 64
_NC = 2
_NS = 16
_NW = _NC * _NS
_L = 16


def _deg_body(npad, rpt, col_hbm, ew_hbm, deg_out, colbuf, ewbuf, zbuf, sdeg):
    c = lax.axis_index("c")
    s = lax.axis_index("s")
    wid = c * _NS + s
    slice_n = npad // _NS

    @pl.loop(0, slice_n // _L)
    def _(i):
        zbuf[pl.ds(i * _L, _L)] = jnp.zeros((_L,), jnp.float32)

    pltpu.sync_copy(zbuf, sdeg.at[pl.ds(s * slice_n, slice_n)])
    r0 = wid * rpt
    pltpu.sync_copy(col_hbm.at[pl.ds(r0, rpt)], colbuf)
    pltpu.sync_copy(ew_hbm.at[pl.ds(r0, rpt)], ewbuf)
    plsc.subcore_barrier()

    @pl.loop(0, rpt)
    def _(j):
        pltpu.sync_copy(ewbuf.at[j], sdeg.at[colbuf.at[j]], add=True)

    plsc.subcore_barrier()
    pltpu.sync_copy(sdeg.at[pl.ds(s * slice_n, slice_n)],
                    deg_out.at[pl.ds(c * npad + s * slice_n, slice_n)])


def _b_body(npad, npadx, rpt, row_hbm, col_hbm, ew_hbm, dinv_hbm, batch_hbm,
            b_out, rowbuf, colbuf, ewbuf, dinvbuf, batchbuf, idxstage,
            valstage, zbuf, sB):
    c = lax.axis_index("c")
    s = lax.axis_index("s")
    wid = c * _NS + s
    bflat = _G * npadx
    slice_b = bflat // _NS
    zn = zbuf.shape[0]

    @pl.loop(0, zn // _L)
    def _(i):
        zbuf[pl.ds(i * _L, _L)] = jnp.zeros((_L,), jnp.float32)

    @pl.loop(0, slice_b // zn)
    def _(t):
        pltpu.sync_copy(zbuf, sB.at[pl.ds(s * slice_b + t * zn, zn)])

    pltpu.sync_copy(dinv_hbm, dinvbuf)
    pltpu.sync_copy(batch_hbm, batchbuf)
    r0 = wid * rpt
    pltpu.sync_copy(row_hbm.at[pl.ds(r0, rpt)], rowbuf)
    pltpu.sync_copy(col_hbm.at[pl.ds(r0, rpt)], colbuf)
    pltpu.sync_copy(ew_hbm.at[pl.ds(r0, rpt)], ewbuf)
    plsc.subcore_barrier()

    @pl.loop(0, rpt)
    def _(j):
        @pl.loop(0, 128 // _L)
        def _(k):
            sl = pl.ds(k * _L, _L)
            r16 = rowbuf[j, sl]
            c16 = colbuf[j, sl]
            w16 = ewbuf[j, sl]
            dr = plsc.load_gather(dinvbuf, [r16])
            dc = plsc.load_gather(dinvbuf, [c16])
            g16 = plsc.load_gather(batchbuf, [c16])
            idxstage[sl] = g16 * npadx + r16
            valstage[sl] = dr * w16 * dc

        pltpu.sync_copy(valstage, sB.at[idxstage], add=True)

    nrows = npad // 128
    @pl.loop(0, (nrows + _NW - 1) // _NW)
    def _(t):
        row = t * _NW + wid

        @pl.when(row < nrows)
        def _():
            @pl.loop(0, 128 // _L)
            def _(k):
                sl = pl.ds(k * _L, _L)
                g16 = batchbuf[pl.ds(row * 128 + k * _L, _L)]
                pad = g16 < 0
                g_safe = jnp.where(pad, 0, g16)
                idxstage[sl] = g_safe * npadx + npad + jnp.where(pad, 1, 0)
                valstage[sl] = jnp.where(pad, 0.0, 1.0)

            pltpu.sync_copy(valstage, sB.at[idxstage], add=True)

    plsc.subcore_barrier()
    pltpu.sync_copy(sB.at[pl.ds(s * slice_b, slice_b)],
                    b_out.at[pl.ds(c * bflat + s * slice_b, slice_b)])


def _dinv_tc_body(deg_ref, out_ref):
    half = out_ref.shape[0]
    d = deg_ref[0:half, :] + deg_ref[half:2 * half, :]
    safe = jnp.where(d > 0, d, 1.0)
    out_ref[...] = jnp.where(d > 0, lax.rsqrt(safe), 0.0)


def _h_tc_body(npad, npadx, x_ref, w_ref, zb_ref, out_ref):
    out_ref[0:npad, :] = jnp.dot(x_ref[...], w_ref[...],
                                 preferred_element_type=jnp.float32)
    out_ref[npad:npadx, :] = zb_ref[...]


def _out_tc_body(b_ref, h_ref, w2_ref, b2_ref, out_ref):
    bsum = b_ref[0] + b_ref[1]
    pooled = jnp.dot(bsum, h_ref[...], preferred_element_type=jnp.float32)
    out_ref[...] = lax.dot_general(
        pooled, w2_ref[...], (((1,), (1,)), ((), ())),
        preferred_element_type=jnp.float32) + b2_ref[...]


def kernel(node_input, edge_input, edge_index, batch, W1, b1, W2, b2):
    n, d_in = node_input.shape
    e = edge_input.shape[0]
    d_out = W2.shape[0]

    npad = ((n + 2047) // 2048) * 2048
    npadx = npad + 128
    e2 = e + n
    echunk = _NW * 128 * 8
    epad = ((e2 + echunk - 1) // echunk) * echunk
    rpt = epad // 128 // _NW

    f32, i32 = jnp.float32, jnp.int32
    loop_idx = jnp.arange(n, dtype=i32)
    row2 = jnp.concatenate([edge_index[0].astype(i32), loop_idx,
                            jnp.zeros((epad - e2,), i32)]).reshape(epad // 128, 128)
    col2 = jnp.concatenate([edge_index[1].astype(i32), loop_idx,
                            jnp.zeros((epad - e2,), i32)]).reshape(epad // 128, 128)
    ew2 = jnp.concatenate([edge_input.astype(f32), jnp.ones((n,), f32),
                           jnp.zeros((epad - e2,), f32)]).reshape(epad // 128, 128)
    batch_pad = jnp.concatenate([batch.astype(i32),
                                 jnp.full((npad - n,), -1, i32)])
    xp = jnp.pad(node_input.astype(f32), ((0, npad - n), (0, 0)))
    zb = jnp.concatenate([b1.astype(f32).reshape(1, d_in),
                          jnp.zeros((127, d_in), f32)])

    mesh = plsc.VectorSubcoreMesh(core_axis_name="c", subcore_axis_name="s")

    deg_kernel = pl.kernel(
        functools.partial(_deg_body, npad, rpt),
        out_type=jax.ShapeDtypeStruct((_NC * npad,), f32),
        mesh=mesh,
        scratch_types=[
            pltpu.VMEM((rpt, 128), i32),
            pltpu.VMEM((rpt, 128), f32),
            pltpu.VMEM((npad // _NS,), f32),
            pltpu.VMEM_SHARED((npad,), f32),
        ],
    )
    deg2 = deg_kernel(col2, ew2)

    dinv = pl.pallas_call(
        _dinv_tc_body,
        out_shape=jax.ShapeDtypeStruct((npad // 128, 128), f32),
    )(deg2.reshape(_NC * npad // 128, 128)).reshape(npad)
    del deg2

    hx = pl.pallas_call(
        functools.partial(_h_tc_body, npad, npadx),
        out_shape=jax.ShapeDtypeStruct((npadx, d_in), f32),
    )(xp, W1.astype(f32), zb)

    cp = pltpu.CompilerParams()
    if "needs_layout_passes" in pltpu.CompilerParams.__dataclass_fields__:
        cp = dataclasses.replace(cp, needs_layout_passes=False)
    b_kernel = pl.kernel(
        functools.partial(_b_body, npad, npadx, rpt),
        out_type=jax.ShapeDtypeStruct((_NC * _G * npadx,), f32),
        mesh=mesh,
        compiler_params=cp,
        scratch_types=[
            pltpu.VMEM((rpt, 128), i32),
            pltpu.VMEM((rpt, 128), i32),
            pltpu.VMEM((rpt, 128), f32),
            pltpu.VMEM((npad,), f32),
            pltpu.VMEM((npad,), i32),
            pltpu.VMEM((128,), i32),
            pltpu.VMEM((128,), f32),
            pltpu.VMEM((2048,), f32),
            pltpu.VMEM_SHARED((_G * npadx,), f32),
        ],
    )
    b2part = b_kernel(row2, col2, ew2, dinv, batch_pad)

    out = pl.pallas_call(
        _out_tc_body,
        out_shape=jax.ShapeDtypeStruct((_G, d_out), f32),
    )(b2part.reshape(_NC, _G, npadx), hx, W2.astype(f32),
      b2.astype(f32).reshape(1, d_out))
    return out
